# Initial kernel scaffold; baseline (speedup 1.0000x reference)
import jax
import jax.numpy as jnp
from jax.experimental import pallas as pl


def kernel(distances, edge_features, residues, node_features, senders, receivers, node_graph_ids, emb_table, We1, be1, We2, be2, Wn1, bn1, Wn2, bn2, W_e, W_s, b_e, W_n, W_in, b_n, W_g, b_g, W_no, b_no):
    raise NotImplementedError("write your pallas kernel here")


# speedup vs baseline: 3.3494x; 3.3494x over previous
import functools

import jax
import jax.numpy as jnp
from jax import lax
from jax.experimental import pallas as pl
from jax.experimental.pallas import tpu as pltpu
from jax.experimental.pallas import tpu_sc as plsc

N_NODES = 10000
N_EDGES = 320000
N_GRAPHS = 32
RBF_SIZE = 16
MAX_DIST = 20.0

NC = 2
NS = 16
NW = NC * NS
EW_PER = N_EDGES // NW
CH = 80
N_CH = EW_PER // CH
NP = 10240
ROWS_PER = NP // NS
AW = 80


def _sigmoid(x):
    return 1.0 / (1.0 + jnp.exp(-x))


def _edge_body(d_ref, ef_ref, We1_ref, be1_ref, We2_ref, be2_ref,
               We_ref, be_ref, out_ref):
    d = d_ref[...]
    centers = lax.broadcasted_iota(jnp.int32, (1, RBF_SIZE), 1).astype(
        jnp.float32) * (MAX_DIST / (RBF_SIZE - 1))
    rbf = jnp.exp(-(d - centers) ** 2)
    e = jnp.concatenate([rbf, ef_ref[...]], axis=1)
    h = jnp.dot(e, We1_ref[...], preferred_element_type=jnp.float32)
    h = jnp.maximum(h + be1_ref[...], 0.0)
    h = jnp.dot(h, We2_ref[...], preferred_element_type=jnp.float32)
    h = jnp.maximum(h + be2_ref[...], 0.0)
    ew = jnp.dot(h, We_ref[...], preferred_element_type=jnp.float32)
    out_ref[...] = ew + be_ref[...]


def _edge_encoder(distances, edge_features, We1, be1, We2, be2, W_e, b_e):
    BE = 8000
    grid = (N_EDGES // BE,)
    full = lambda shape: pl.BlockSpec(shape, lambda i: (0, 0))
    return pl.pallas_call(
        _edge_body,
        grid=grid,
        in_specs=[
            pl.BlockSpec((BE, 1), lambda i: (i, 0)),
            pl.BlockSpec((BE, 16), lambda i: (i, 0)),
            full((32, 4)), full((1, 4)), full((4, 8)), full((1, 8)),
            full((8, 64)), full((1, 64)),
        ],
        out_specs=pl.BlockSpec((BE, 64), lambda i: (i, 0)),
        out_shape=jax.ShapeDtypeStruct((N_EDGES, 64), jnp.float32),
    )(distances.reshape(N_EDGES, 1), edge_features,
      We1, be1.reshape(1, 4), We2, be2.reshape(1, 8),
      W_e, b_e.reshape(1, 64))


def _node_body(res_ref, nf_ref, emb_ref, Wn1_ref, bn1_ref, Wn2_ref, bn2_ref,
               Ws_ref, x2_ref, xs_ref):
    r = res_ref[...]
    oh = (r == lax.broadcasted_iota(jnp.int32, (1, 22), 1)).astype(
        jnp.float32)
    emb = jnp.dot(oh, emb_ref[...], preferred_element_type=jnp.float32)
    x = jnp.concatenate([emb, nf_ref[...]], axis=1)
    h = jnp.dot(x, Wn1_ref[...], preferred_element_type=jnp.float32)
    h = jnp.maximum(h + bn1_ref[...], 0.0)
    h = jnp.dot(h, Wn2_ref[...], preferred_element_type=jnp.float32)
    x2 = jnp.maximum(h + bn2_ref[...], 0.0)
    x2_ref[...] = x2
    xs = jnp.dot(x2, Ws_ref[...], preferred_element_type=jnp.float32)
    bn = xs.shape[0]
    pad = jnp.concatenate(
        [jnp.full((bn, 1), 0.5, jnp.float32),
         jnp.zeros((bn, AW - 65), jnp.float32)], axis=1)
    xs_ref[...] = jnp.concatenate([xs, pad], axis=1)


def _node_encoder(residues, node_features, emb_table, Wn1, bn1, Wn2, bn2, W_s):
    BN = 2000
    grid = (N_NODES // BN,)
    full = lambda shape: pl.BlockSpec(shape, lambda i: (0, 0))
    return pl.pallas_call(
        _node_body,
        grid=grid,
        in_specs=[
            pl.BlockSpec((BN, 1), lambda i: (i, 0)),
            pl.BlockSpec((BN, 96), lambda i: (i, 0)),
            full((22, 32)), full((128, 8)), full((1, 8)),
            full((8, 16)), full((1, 16)), full((16, 64)),
        ],
        out_specs=[
            pl.BlockSpec((BN, 16), lambda i: (i, 0)),
            pl.BlockSpec((BN, AW), lambda i: (i, 0)),
        ],
        out_shape=[
            jax.ShapeDtypeStruct((N_NODES, 16), jnp.float32),
            jax.ShapeDtypeStruct((N_NODES, AW), jnp.float32),
        ],
    )(residues.reshape(N_NODES, 1), node_features,
      emb_table, Wn1, bn1.reshape(1, 8), Wn2, bn2.reshape(1, 16), W_s)


def _sc_body(ew_hbm, xs_hbm, snd_hbm, rcv_hbm, agg_out,
             agg_sh, idx_s, idx_r, ewb, gb, sem):
    c = lax.axis_index("c")
    s = lax.axis_index("s")
    wid = c * NS + s
    row0 = s * ROWS_PER

    @pl.loop(0, CH)
    def _zero_gb(r):
        for j in range(AW // 16):
            gb[r, pl.ds(16 * j, 16)] = jnp.zeros((16,), jnp.float32)

    for k in range(ROWS_PER // CH):
        pltpu.sync_copy(gb, agg_sh.at[pl.ds(row0 + k * CH, CH)])

    plsc.subcore_barrier()

    def _direction(idx_from, idx_to):
        pltpu.async_copy(xs_hbm.at[idx_from], gb, sem).wait()

        @pl.loop(0, CH)
        def _fuse(r):
            for j in range(4):
                sl = pl.ds(16 * j, 16)
                gb[r, sl] = jnp.maximum(ewb[r, sl] + gb[r, sl], 0.0)
            sl = pl.ds(64, 16)
            gb[r, sl] = gb[r, sl] * 2.0

        pltpu.sync_copy(gb, agg_sh.at[idx_to], add=True)

    @pl.loop(0, N_CH)
    def _chunk(k):
        base = wid * EW_PER + k * CH
        pltpu.sync_copy(snd_hbm.at[pl.ds(base, CH)], idx_s)
        pltpu.sync_copy(rcv_hbm.at[pl.ds(base, CH)], idx_r)
        pltpu.sync_copy(ew_hbm.at[pl.ds(base, CH)], ewb)
        _direction(idx_s, idx_r)
        _direction(idx_r, idx_s)

    plsc.subcore_barrier()
    pltpu.sync_copy(agg_sh.at[pl.ds(row0, ROWS_PER)],
                    agg_out.at[c, pl.ds(row0, ROWS_PER)])


def _sc_aggregate(ew, xs_pad, senders, receivers):
    mesh = plsc.VectorSubcoreMesh(core_axis_name="c", subcore_axis_name="s")
    return pl.kernel(
        _sc_body,
        out_type=jax.ShapeDtypeStruct((NC, NP, AW), jnp.float32),
        mesh=mesh,
        compiler_params=pltpu.CompilerParams(use_tc_tiling_on_sc=False),
        scratch_types=[
            pltpu.VMEM_SHARED((NP, AW), jnp.float32),
            pltpu.VMEM((CH,), jnp.int32),
            pltpu.VMEM((CH,), jnp.int32),
            pltpu.VMEM((CH, 64), jnp.float32),
            pltpu.VMEM((CH, AW), jnp.float32),
            pltpu.SemaphoreType.DMA,
        ],
    )(ew, xs_pad, senders, receivers)


def _epi_body(x2_ref, ap_ref, gid_ref, Wn_ref, Win_ref, bn_ref,
              Wg_ref, bg_ref, Wno_ref, bno_ref, no_ref, go_ref):
    asum = ap_ref[0] + ap_ref[1]
    deg = jnp.maximum(asum[:, 64:65], 1.0)
    agg = asum[:, :64] / deg
    x2 = x2_ref[...]
    x3 = jnp.dot(x2, Wn_ref[...], preferred_element_type=jnp.float32)
    x3 = x3 + jnp.dot(agg, Win_ref[...], preferred_element_type=jnp.float32)
    x3 = jnp.maximum(x3 + bn_ref[...], 0.0)
    no_ref[...] = _sigmoid(
        jnp.dot(x3, Wno_ref[...], preferred_element_type=jnp.float32)
        + bno_ref[...])
    oh = (gid_ref[...] ==
          lax.broadcasted_iota(jnp.int32, (N_GRAPHS, N_NODES), 0)
          ).astype(jnp.float32)
    gsum = jnp.dot(oh, x3, preferred_element_type=jnp.float32)
    nper = jnp.maximum(jnp.sum(oh, axis=1, keepdims=True), 1.0)
    gmean = gsum / nper
    go_ref[...] = _sigmoid(
        jnp.dot(gmean, Wg_ref[...], preferred_element_type=jnp.float32)
        + bg_ref[...])


def _epilog(x2, agg_parts, node_graph_ids, W_n, W_in, b_n, W_g, b_g,
            W_no, b_no):
    full = lambda shape: pl.BlockSpec(shape, lambda i: tuple(0 for _ in shape))
    return pl.pallas_call(
        _epi_body,
        grid=(1,),
        in_specs=[
            full((N_NODES, 16)),
            full((NC, N_NODES, AW)),
            full((1, N_NODES)),
            full((16, 128)), full((64, 128)), full((1, 128)),
            full((128, 2)), full((1, 2)), full((128, 2)), full((1, 2)),
        ],
        out_specs=[
            full((N_NODES, 2)),
            full((N_GRAPHS, 2)),
        ],
        out_shape=[
            jax.ShapeDtypeStruct((N_NODES, 2), jnp.float32),
            jax.ShapeDtypeStruct((N_GRAPHS, 2), jnp.float32),
        ],
    )(x2, agg_parts, node_graph_ids.reshape(1, N_NODES),
      W_n, W_in, b_n.reshape(1, 128), W_g, b_g.reshape(1, 2),
      W_no, b_no.reshape(1, 2))


def kernel(distances, edge_features, residues, node_features, senders,
           receivers, node_graph_ids, emb_table, We1, be1, We2, be2,
           Wn1, bn1, Wn2, bn2, W_e, W_s, b_e, W_n, W_in, b_n, W_g, b_g,
           W_no, b_no):
    senders = jnp.asarray(senders, jnp.int32)
    receivers = jnp.asarray(receivers, jnp.int32)
    residues = jnp.asarray(residues, jnp.int32)
    node_graph_ids = jnp.asarray(node_graph_ids, jnp.int32)

    ew = _edge_encoder(distances, edge_features, We1, be1, We2, be2, W_e, b_e)
    x2, xs_pad = _node_encoder(residues, node_features, emb_table,
                               Wn1, bn1, Wn2, bn2, W_s)
    xs_pad = jnp.pad(xs_pad, ((0, NP - N_NODES), (0, 0)))
    agg_parts = _sc_aggregate(ew, xs_pad, senders, receivers)
    node_out, global_out = _epilog(x2, agg_parts, node_graph_ids,
                                   W_n, W_in, b_n, W_g, b_g, W_no, b_no)
    return (node_out, global_out)


# speedup vs baseline: 4.3225x; 1.2905x over previous
import functools

import jax
import jax.numpy as jnp
from jax import lax
from jax.experimental import pallas as pl
from jax.experimental.pallas import tpu as pltpu
from jax.experimental.pallas import tpu_sc as plsc

N_NODES = 10000
N_EDGES = 320000
N_GRAPHS = 32
RBF_SIZE = 16
MAX_DIST = 20.0

NC = 2
NS = 16
NW = NC * NS
EW_PER = N_EDGES // NW
CH = 80
N_CH = EW_PER // CH
NP = 10240
ROWS_PER = NP // NS
AW = 64


def _sigmoid(x):
    return 1.0 / (1.0 + jnp.exp(-x))


def _edge_body(d_ref, ef_ref, We1_ref, be1_ref, We2_ref, be2_ref,
               We_ref, be_ref, out_ref):
    d = d_ref[...]
    centers = lax.broadcasted_iota(jnp.int32, (1, RBF_SIZE), 1).astype(
        jnp.float32) * (MAX_DIST / (RBF_SIZE - 1))
    rbf = jnp.exp(-(d - centers) ** 2)
    e = jnp.concatenate([rbf, ef_ref[...]], axis=1)
    h = jnp.dot(e, We1_ref[...], preferred_element_type=jnp.float32)
    h = jnp.maximum(h + be1_ref[...], 0.0)
    h = jnp.dot(h, We2_ref[...], preferred_element_type=jnp.float32)
    h = jnp.maximum(h + be2_ref[...], 0.0)
    ew = jnp.dot(h, We_ref[...], preferred_element_type=jnp.float32)
    out_ref[...] = ew + be_ref[...]


def _edge_encoder(distances, edge_features, We1, be1, We2, be2, W_e, b_e):
    BE = 8000
    grid = (N_EDGES // BE,)
    full = lambda shape: pl.BlockSpec(shape, lambda i: (0, 0))
    return pl.pallas_call(
        _edge_body,
        grid=grid,
        in_specs=[
            pl.BlockSpec((BE, 1), lambda i: (i, 0)),
            pl.BlockSpec((BE, 16), lambda i: (i, 0)),
            full((32, 4)), full((1, 4)), full((4, 8)), full((1, 8)),
            full((8, 64)), full((1, 64)),
        ],
        out_specs=pl.BlockSpec((BE, 64), lambda i: (i, 0)),
        out_shape=jax.ShapeDtypeStruct((N_EDGES, 64), jnp.float32),
    )(distances.reshape(N_EDGES, 1), edge_features,
      We1, be1.reshape(1, 4), We2, be2.reshape(1, 8),
      W_e, b_e.reshape(1, 64))


def _node_body(res_ref, nf_ref, emb_ref, Wn1_ref, bn1_ref, Wn2_ref, bn2_ref,
               Ws_ref, x2_ref, xs_ref):
    r = res_ref[...]
    oh = (r == lax.broadcasted_iota(jnp.int32, (1, 22), 1)).astype(
        jnp.float32)
    emb = jnp.dot(oh, emb_ref[...], preferred_element_type=jnp.float32)
    x = jnp.concatenate([emb, nf_ref[...]], axis=1)
    h = jnp.dot(x, Wn1_ref[...], preferred_element_type=jnp.float32)
    h = jnp.maximum(h + bn1_ref[...], 0.0)
    h = jnp.dot(h, Wn2_ref[...], preferred_element_type=jnp.float32)
    x2 = jnp.maximum(h + bn2_ref[...], 0.0)
    x2_ref[...] = x2
    xs_ref[...] = jnp.dot(x2, Ws_ref[...],
                          preferred_element_type=jnp.float32)


def _node_encoder(residues, node_features, emb_table, Wn1, bn1, Wn2, bn2, W_s):
    BN = 2000
    grid = (N_NODES // BN,)
    full = lambda shape: pl.BlockSpec(shape, lambda i: (0, 0))
    return pl.pallas_call(
        _node_body,
        grid=grid,
        in_specs=[
            pl.BlockSpec((BN, 1), lambda i: (i, 0)),
            pl.BlockSpec((BN, 96), lambda i: (i, 0)),
            full((22, 32)), full((128, 8)), full((1, 8)),
            full((8, 16)), full((1, 16)), full((16, 64)),
        ],
        out_specs=[
            pl.BlockSpec((BN, 16), lambda i: (i, 0)),
            pl.BlockSpec((BN, AW), lambda i: (i, 0)),
        ],
        out_shape=[
            jax.ShapeDtypeStruct((N_NODES, 16), jnp.float32),
            jax.ShapeDtypeStruct((N_NODES, AW), jnp.float32),
        ],
    )(residues.reshape(N_NODES, 1), node_features,
      emb_table, Wn1, bn1.reshape(1, 8), Wn2, bn2.reshape(1, 16), W_s)


def _sc_body(ew_hbm, xs_hbm, snd_hbm, rcv_hbm, agg_out, deg_out,
             agg_sh, idx_s, idx_r, ewb, gb, deg, sem):
    c = lax.axis_index("c")
    s = lax.axis_index("s")
    wid = c * NS + s
    row0 = s * ROWS_PER
    ones16 = jnp.ones((16,), jnp.float32)

    @pl.loop(0, CH)
    def _zero_gb(r):
        for j in range(AW // 16):
            gb[r, pl.ds(16 * j, 16)] = jnp.zeros((16,), jnp.float32)

    for k in range(ROWS_PER // CH):
        pltpu.sync_copy(gb, agg_sh.at[pl.ds(row0 + k * CH, CH)])

    @pl.loop(0, NP // 16)
    def _zero_deg(i):
        deg[pl.ds(i * 16, 16)] = jnp.zeros((16,), jnp.float32)

    plsc.subcore_barrier()

    def _direction(idx_from, idx_to):
        pltpu.async_copy(xs_hbm.at[idx_from], gb, sem).wait()

        @pl.loop(0, CH)
        def _fuse(r):
            for j in range(AW // 16):
                sl = pl.ds(16 * j, 16)
                gb[r, sl] = jnp.maximum(ewb[r, sl] + gb[r, sl], 0.0)

        pltpu.sync_copy(gb, agg_sh.at[idx_to], add=True)
        for j in range(CH // 16):
            iv = idx_to[pl.ds(16 * j, 16)]
            plsc.addupdate_scatter(deg, [iv], ones16)

    @pl.loop(0, N_CH)
    def _chunk(k):
        base = wid * EW_PER + k * CH
        pltpu.sync_copy(snd_hbm.at[pl.ds(base, CH)], idx_s)
        pltpu.sync_copy(rcv_hbm.at[pl.ds(base, CH)], idx_r)
        pltpu.sync_copy(ew_hbm.at[pl.ds(base, CH)], ewb)
        _direction(idx_s, idx_r)
        _direction(idx_r, idx_s)

    plsc.subcore_barrier()
    pltpu.sync_copy(agg_sh.at[pl.ds(row0, ROWS_PER)],
                    agg_out.at[c, pl.ds(row0, ROWS_PER)])
    pltpu.sync_copy(deg, deg_out.at[wid])


def _sc_aggregate(ew, xs_pad, senders, receivers):
    mesh = plsc.VectorSubcoreMesh(core_axis_name="c", subcore_axis_name="s")
    return pl.kernel(
        _sc_body,
        out_type=[
            jax.ShapeDtypeStruct((NC, NP, AW), jnp.float32),
            jax.ShapeDtypeStruct((NW, NP), jnp.float32),
        ],
        mesh=mesh,
        compiler_params=pltpu.CompilerParams(use_tc_tiling_on_sc=False,
                                             needs_layout_passes=False),
        scratch_types=[
            pltpu.VMEM_SHARED((NP, AW), jnp.float32),
            pltpu.VMEM((CH,), jnp.int32),
            pltpu.VMEM((CH,), jnp.int32),
            pltpu.VMEM((CH, 64), jnp.float32),
            pltpu.VMEM((CH, AW), jnp.float32),
            pltpu.VMEM((NP,), jnp.float32),
            pltpu.SemaphoreType.DMA,
        ],
    )(ew, xs_pad, senders, receivers)


def _epi_body(x2_ref, ap_ref, dp_ref, gid_ref, Wn_ref, Win_ref, bn_ref,
              Wg_ref, bg_ref, Wno_ref, bno_ref, no_ref, go_ref):
    asum = ap_ref[0] + ap_ref[1]
    deg = lax.dot_general(dp_ref[...], jnp.ones((NW, 1), jnp.float32),
                          (((0,), (0,)), ((), ())),
                          preferred_element_type=jnp.float32)
    deg = jnp.maximum(deg[:N_NODES], 1.0)
    agg = asum / deg
    x2 = x2_ref[...]
    x3 = jnp.dot(x2, Wn_ref[...], preferred_element_type=jnp.float32)
    x3 = x3 + jnp.dot(agg, Win_ref[...], preferred_element_type=jnp.float32)
    x3 = jnp.maximum(x3 + bn_ref[...], 0.0)
    no_ref[...] = _sigmoid(
        jnp.dot(x3, Wno_ref[...], preferred_element_type=jnp.float32)
        + bno_ref[...])
    oh = (gid_ref[...] ==
          lax.broadcasted_iota(jnp.int32, (N_GRAPHS, N_NODES), 0)
          ).astype(jnp.float32)
    gsum = jnp.dot(oh, x3, preferred_element_type=jnp.float32)
    nper = jnp.maximum(jnp.sum(oh, axis=1, keepdims=True), 1.0)
    gmean = gsum / nper
    go_ref[...] = _sigmoid(
        jnp.dot(gmean, Wg_ref[...], preferred_element_type=jnp.float32)
        + bg_ref[...])


def _epilog(x2, agg_parts, deg_parts, node_graph_ids, W_n, W_in, b_n,
            W_g, b_g, W_no, b_no):
    full = lambda shape: pl.BlockSpec(shape, lambda i: tuple(0 for _ in shape))
    return pl.pallas_call(
        _epi_body,
        grid=(1,),
        in_specs=[
            full((N_NODES, 16)),
            full((NC, N_NODES, AW)),
            full((NW, NP)),
            full((1, N_NODES)),
            full((16, 128)), full((64, 128)), full((1, 128)),
            full((128, 2)), full((1, 2)), full((128, 2)), full((1, 2)),
        ],
        out_specs=[
            full((N_NODES, 2)),
            full((N_GRAPHS, 2)),
        ],
        out_shape=[
            jax.ShapeDtypeStruct((N_NODES, 2), jnp.float32),
            jax.ShapeDtypeStruct((N_GRAPHS, 2), jnp.float32),
        ],
    )(x2, agg_parts, deg_parts, node_graph_ids.reshape(1, N_NODES),
      W_n, W_in, b_n.reshape(1, 128), W_g, b_g.reshape(1, 2),
      W_no, b_no.reshape(1, 2))


def kernel(distances, edge_features, residues, node_features, senders,
           receivers, node_graph_ids, emb_table, We1, be1, We2, be2,
           Wn1, bn1, Wn2, bn2, W_e, W_s, b_e, W_n, W_in, b_n, W_g, b_g,
           W_no, b_no):
    senders = jnp.asarray(senders, jnp.int32)
    receivers = jnp.asarray(receivers, jnp.int32)
    residues = jnp.asarray(residues, jnp.int32)
    node_graph_ids = jnp.asarray(node_graph_ids, jnp.int32)

    ew = _edge_encoder(distances, edge_features, We1, be1, We2, be2, W_e, b_e)
    x2, xs_pad = _node_encoder(residues, node_features, emb_table,
                               Wn1, bn1, Wn2, bn2, W_s)
    xs_pad = jnp.pad(xs_pad, ((0, NP - N_NODES), (0, 0)))
    agg_parts, deg_parts = _sc_aggregate(ew, xs_pad, senders, receivers)
    node_out, global_out = _epilog(x2, agg_parts, deg_parts, node_graph_ids,
                                   W_n, W_in, b_n, W_g, b_g, W_no, b_no)
    return (node_out, global_out)
